# baseline (speedup 1.0000x reference)
import jax
import jax.numpy as jnp
from jax import lax
from jax.experimental import pallas as pl
from jax.experimental.pallas import tpu as pltpu

_K = 2
_BLK = 1024
_NBUF = 4


def _scale_kernel(x_hbm, w_hbm, o_hbm, xb, wb, ob, in_sems, w_sems, out_sems):
    n = x_hbm.shape[0]
    nsteps = n // _BLK

    def in_copy(step, slot):
        return pltpu.make_async_copy(
            x_hbm.at[pl.ds(step * _BLK, _BLK), :], xb.at[slot],
            in_sems.at[slot])

    def w_copy(step, slot):
        return pltpu.make_async_copy(
            w_hbm.at[pl.ds(step * _BLK, _BLK), :], wb.at[slot],
            w_sems.at[slot])

    def out_copy(step, slot):
        return pltpu.make_async_copy(
            ob.at[slot], o_hbm.at[pl.ds(step * _BLK, _BLK), :],
            out_sems.at[slot])

    for s in range(_NBUF):
        in_copy(s, s).start()
        w_copy(s, s).start()

    def step_body(i, carry):
        slot = lax.rem(i, _NBUF)
        in_copy(i, slot).wait()
        w_copy(i, slot).wait()

        @pl.when(i >= _NBUF)
        def _():
            out_copy(i - _NBUF, slot).wait()

        w = wb[slot]
        scale = 1.0 + jnp.sum(w, axis=1, keepdims=True)
        ob[slot] = xb[slot] * scale
        out_copy(i, slot).start()

        @pl.when(i + _NBUF < nsteps)
        def _():
            in_copy(i + _NBUF, slot).start()
            w_copy(i + _NBUF, slot).start()

        return carry

    lax.fori_loop(0, nsteps, step_body, 0)
    for s in range(_NBUF):
        i = nsteps - _NBUF + s
        out_copy(i, i % _NBUF).wait()


def kernel(x, weights, indices, expert_offsets, expert_token_counts):
    C, S, D = x.shape
    n = C * S
    xf = x.reshape(n, D)
    wf = weights.reshape(n, _K)
    out = pl.pallas_call(
        _scale_kernel,
        in_specs=[
            pl.BlockSpec(memory_space=pl.ANY),
            pl.BlockSpec(memory_space=pl.ANY),
        ],
        out_specs=pl.BlockSpec(memory_space=pl.ANY),
        out_shape=jax.ShapeDtypeStruct((n, D), x.dtype),
        scratch_shapes=[
            pltpu.VMEM((_NBUF, _BLK, D), jnp.float32),
            pltpu.VMEM((_NBUF, _BLK, _K), jnp.float32),
            pltpu.VMEM((_NBUF, _BLK, D), jnp.float32),
            pltpu.SemaphoreType.DMA((_NBUF,)),
            pltpu.SemaphoreType.DMA((_NBUF,)),
            pltpu.SemaphoreType.DMA((_NBUF,)),
        ],
    )(xf, wf)
    return out.reshape(C, S, D)


# speedup vs baseline: 1.0163x; 1.0163x over previous
import jax
import jax.numpy as jnp
from jax import lax
from jax.experimental import pallas as pl
from jax.experimental.pallas import tpu as pltpu

_K = 2
_BLK = 2048
_NBUF = 4


def _scale_kernel(x_hbm, w_hbm, o_hbm, xb, wb, in_sems, w_sems, out_sems):
    n = x_hbm.shape[0]
    nsteps = n // _BLK

    def in_copy(step, slot):
        return pltpu.make_async_copy(
            x_hbm.at[pl.ds(step * _BLK, _BLK), :], xb.at[slot],
            in_sems.at[slot])

    def w_copy(step, slot):
        return pltpu.make_async_copy(
            w_hbm.at[pl.ds(step * _BLK, _BLK), :], wb.at[slot],
            w_sems.at[slot])

    def out_copy(step, slot):
        return pltpu.make_async_copy(
            xb.at[slot], o_hbm.at[pl.ds(step * _BLK, _BLK), :],
            out_sems.at[slot])

    for s in range(_NBUF):
        in_copy(s, s).start()
        w_copy(s, s).start()

    def step_body(i, carry):
        slot = lax.rem(i, _NBUF)
        in_copy(i, slot).wait()
        w_copy(i, slot).wait()

        @pl.when(i >= _NBUF)
        def _():
            out_copy(i - _NBUF, slot).wait()

        w = wb[slot]
        scale = 1.0 + jnp.sum(w, axis=1, keepdims=True)
        xb[slot] = xb[slot] * scale
        out_copy(i, slot).start()

        @pl.when(i + _NBUF < nsteps)
        def _():
            in_copy(i + _NBUF, slot).start()
            w_copy(i + _NBUF, slot).start()

        return carry

    lax.fori_loop(0, nsteps, step_body, 0)
    for s in range(_NBUF):
        i = nsteps - _NBUF + s
        out_copy(i, i % _NBUF).wait()


def kernel(x, weights, indices, expert_offsets, expert_token_counts):
    C, S, D = x.shape
    n = C * S
    xf = x.reshape(n, D)
    wf = weights.reshape(n, _K)
    out = pl.pallas_call(
        _scale_kernel,
        in_specs=[
            pl.BlockSpec(memory_space=pl.ANY),
            pl.BlockSpec(memory_space=pl.ANY),
        ],
        out_specs=pl.BlockSpec(memory_space=pl.ANY),
        out_shape=jax.ShapeDtypeStruct((n, D), x.dtype),
        scratch_shapes=[
            pltpu.VMEM((_NBUF, _BLK, D), jnp.float32),
            pltpu.VMEM((_NBUF, _BLK, _K), jnp.float32),
            pltpu.SemaphoreType.DMA((_NBUF,)),
            pltpu.SemaphoreType.DMA((_NBUF,)),
            pltpu.SemaphoreType.DMA((_NBUF,)),
        ],
    )(xf, wf)
    return out.reshape(C, S, D)


# speedup vs baseline: 1.0480x; 1.0312x over previous
import jax
import jax.numpy as jnp
from jax import lax
from jax.experimental import pallas as pl
from jax.experimental.pallas import tpu as pltpu

_K = 2
_BLK = 4096
_NBUF = 2


def _scale_kernel(x_hbm, w_hbm, o_hbm, xb, wb, in_sems, w_sems, out_sems):
    n = x_hbm.shape[0]
    nsteps = n // _BLK

    def in_copy(step, slot):
        return pltpu.make_async_copy(
            x_hbm.at[pl.ds(step * _BLK, _BLK), :], xb.at[slot],
            in_sems.at[slot])

    def w_copy(step, slot):
        return pltpu.make_async_copy(
            w_hbm.at[pl.ds(step * _BLK, _BLK), :], wb.at[slot],
            w_sems.at[slot])

    def out_copy(step, slot):
        return pltpu.make_async_copy(
            xb.at[slot], o_hbm.at[pl.ds(step * _BLK, _BLK), :],
            out_sems.at[slot])

    for s in range(_NBUF):
        in_copy(s, s).start()
        w_copy(s, s).start()

    def step_body(i, carry):
        slot = lax.rem(i, _NBUF)
        in_copy(i, slot).wait()
        w_copy(i, slot).wait()

        @pl.when(i >= _NBUF)
        def _():
            out_copy(i - _NBUF, slot).wait()

        w = wb[slot]
        scale = 1.0 + jnp.sum(w, axis=1, keepdims=True)
        xb[slot] = xb[slot] * scale
        out_copy(i, slot).start()

        @pl.when(i + _NBUF < nsteps)
        def _():
            in_copy(i + _NBUF, slot).start()
            w_copy(i + _NBUF, slot).start()

        return carry

    lax.fori_loop(0, nsteps, step_body, 0)
    for s in range(_NBUF):
        i = nsteps - _NBUF + s
        out_copy(i, i % _NBUF).wait()


def kernel(x, weights, indices, expert_offsets, expert_token_counts):
    C, S, D = x.shape
    n = C * S
    xf = x.reshape(n, D)
    wf = weights.reshape(n, _K)
    out = pl.pallas_call(
        _scale_kernel,
        in_specs=[
            pl.BlockSpec(memory_space=pl.ANY),
            pl.BlockSpec(memory_space=pl.ANY),
        ],
        out_specs=pl.BlockSpec(memory_space=pl.ANY),
        out_shape=jax.ShapeDtypeStruct((n, D), x.dtype),
        scratch_shapes=[
            pltpu.VMEM((_NBUF, _BLK, D), jnp.float32),
            pltpu.VMEM((_NBUF, _BLK, _K), jnp.float32),
            pltpu.SemaphoreType.DMA((_NBUF,)),
            pltpu.SemaphoreType.DMA((_NBUF,)),
            pltpu.SemaphoreType.DMA((_NBUF,)),
        ],
    )(xf, wf)
    return out.reshape(C, S, D)


# speedup vs baseline: 1.0845x; 1.0348x over previous
import jax
import jax.numpy as jnp
from jax.experimental import pallas as pl

_K = 2
_BLK = 3328


def _scale_kernel(x_ref, w_ref, o_ref):
    w = w_ref[...]
    scale = 1.0 + jnp.sum(w, axis=1, keepdims=True)
    o_ref[...] = x_ref[...] * scale


def kernel(x, weights, indices, expert_offsets, expert_token_counts):
    C, S, D = x.shape
    n = C * S
    xf = x.reshape(n, D)
    wf = weights.reshape(n, _K)
    out = pl.pallas_call(
        _scale_kernel,
        grid=(pl.cdiv(n, _BLK),),
        in_specs=[
            pl.BlockSpec((_BLK, D), lambda i: (i, 0)),
            pl.BlockSpec((_BLK, _K), lambda i: (i, 0)),
        ],
        out_specs=pl.BlockSpec((_BLK, D), lambda i: (i, 0)),
        out_shape=jax.ShapeDtypeStruct((n, D), x.dtype),
    )(xf, wf)
    return out.reshape(C, S, D)
